# Initial kernel scaffold; baseline (speedup 1.0000x reference)
import jax
import jax.numpy as jnp
from jax.experimental import pallas as pl


def kernel(x, w_theta, w_phi, w_g, w_o, gamma):
    raise NotImplementedError("write your pallas kernel here")


# speedup vs baseline: 1.6266x; 1.6266x over previous
import jax
import jax.numpy as jnp
from jax import lax
from jax.experimental import pallas as pl
from jax.experimental.pallas import tpu as pltpu

_QB = 512


def kernel(x, w_theta, w_phi, w_g, w_o, gamma):
    B, C, H, W = x.shape
    N = H * W
    Hp, Wp = H // 2, W // 2
    M = Hp * Wp
    C8 = w_theta.shape[0]
    C2 = w_g.shape[0]
    CP = C8 + C2
    QB = _QB

    x_flat = x.reshape(B, C, N)
    w_pg = jnp.concatenate([w_phi, w_g], axis=0)
    gamma_arr = jnp.reshape(gamma, (1,)).astype(jnp.float32)

    def prep_kernel(x_ref, w_pg_ref, w_th_ref, kg_ref, full_scr, pool_scr):
        xb = x_ref[0]
        full_scr[...] = jnp.dot(
            w_pg_ref[...], xb, preferred_element_type=jnp.float32
        )
        r = lax.broadcasted_iota(jnp.int32, (W, Wp), 0)
        c = lax.broadcasted_iota(jnp.int32, (W, Wp), 1)
        E0 = jnp.where(r == 2 * c, 1.0, 0.0).astype(jnp.float32)
        E1 = jnp.where(r == 2 * c + 1, 1.0, 0.0).astype(jnp.float32)
        for hp in range(Hp):
            a = full_scr[:, (2 * hp) * W:(2 * hp) * W + W]
            b = full_scr[:, (2 * hp + 1) * W:(2 * hp + 1) * W + W]
            hm = jnp.maximum(a, b)
            pooled = jnp.maximum(
                jnp.dot(hm, E0, preferred_element_type=jnp.float32),
                jnp.dot(hm, E1, preferred_element_type=jnp.float32),
            )
            pool_scr[:, hp * Wp:(hp + 1) * Wp] = pooled
        k_eff = lax.dot_general(
            w_th_ref[...], pool_scr[:C8, :],
            (((0,), (0,)), ((), ())),
            preferred_element_type=jnp.float32,
        )
        kg_ref[0, :C, :] = k_eff
        kg_ref[0, C:, :] = pool_scr[C8:, :]

    kg = pl.pallas_call(
        prep_kernel,
        grid=(B,),
        in_specs=[
            pl.BlockSpec((1, C, N), lambda b: (b, 0, 0)),
            pl.BlockSpec((CP, C), lambda b: (0, 0)),
            pl.BlockSpec((C8, C), lambda b: (0, 0)),
        ],
        out_specs=pl.BlockSpec((1, C + C2, M), lambda b: (b, 0, 0)),
        out_shape=jax.ShapeDtypeStruct((B, C + C2, M), jnp.float32),
        scratch_shapes=[
            pltpu.VMEM((CP, N), jnp.float32),
            pltpu.VMEM((CP, M), jnp.float32),
        ],
        compiler_params=pltpu.CompilerParams(
            dimension_semantics=("parallel",),
            vmem_limit_bytes=40 * 1024 * 1024,
        ),
        name="nl_attn_prep",
    )(x_flat, w_pg, w_theta)

    def attn_kernel(x_ref, kg_ref, w_o_ref, gamma_ref, o_ref):
        xb = x_ref[0]
        k_eff = kg_ref[0, :C, :]
        g = kg_ref[0, C:, :]
        scores = lax.dot_general(
            xb, k_eff, (((0,), (0,)), ((), ())),
            preferred_element_type=jnp.float32,
        )
        mx = jnp.max(scores, axis=-1, keepdims=True)
        e = jnp.exp(scores - mx)
        s = jnp.sum(e, axis=-1, keepdims=True)
        beta = e / s
        o_midT = lax.dot_general(
            g, beta, (((1,), (1,)), ((), ())),
            preferred_element_type=jnp.float32,
        )
        oT = jnp.dot(w_o_ref[...], o_midT, preferred_element_type=jnp.float32)
        o_ref[0] = gamma_ref[0] * oT + xb

    out_flat = pl.pallas_call(
        attn_kernel,
        grid=(B, N // QB),
        in_specs=[
            pl.BlockSpec((1, C, QB), lambda b, q: (b, 0, q)),
            pl.BlockSpec((1, C + C2, M), lambda b, q: (b, 0, 0)),
            pl.BlockSpec((C, C2), lambda b, q: (0, 0)),
            pl.BlockSpec(memory_space=pltpu.SMEM),
        ],
        out_specs=pl.BlockSpec((1, C, QB), lambda b, q: (b, 0, q)),
        out_shape=jax.ShapeDtypeStruct((B, C, N), jnp.float32),
        compiler_params=pltpu.CompilerParams(
            dimension_semantics=("parallel", "arbitrary"),
            vmem_limit_bytes=48 * 1024 * 1024,
        ),
        name="nl_attn_main",
    )(x_flat, kg, w_o, gamma_arr)

    return out_flat.reshape(B, C, H, W)


# speedup vs baseline: 2.0711x; 1.2732x over previous
import jax
import jax.numpy as jnp
from jax import lax
from jax.experimental import pallas as pl
from jax.experimental.pallas import tpu as pltpu

_QB = 512
_LOG2E = 1.4426950408889634


def kernel(x, w_theta, w_phi, w_g, w_o, gamma):
    B, C, H, W = x.shape
    N = H * W
    Hp, Wp = H // 2, W // 2
    M = Hp * Wp
    C8 = w_theta.shape[0]
    C2 = w_g.shape[0]
    CP = C8 + C2
    KG = C + C2 + 8
    QB = _QB

    x_flat = x.reshape(B, C, N)
    w_pg = jnp.concatenate([w_phi, w_g], axis=0)
    w_theta_s = (w_theta * _LOG2E).astype(jnp.float32)
    gamma_arr = jnp.reshape(gamma, (1,)).astype(jnp.float32)

    def prep_kernel(x_ref, w_pg_ref, w_th_ref, kg_ref, full_scr, pool_scr):
        xb = x_ref[0]
        full_scr[...] = jnp.dot(
            w_pg_ref[...], xb, preferred_element_type=jnp.float32
        )
        r = lax.broadcasted_iota(jnp.int32, (W, Wp), 0)
        c = lax.broadcasted_iota(jnp.int32, (W, Wp), 1)
        E0 = jnp.where(r == 2 * c, 1.0, 0.0).astype(jnp.float32)
        E1 = jnp.where(r == 2 * c + 1, 1.0, 0.0).astype(jnp.float32)
        for hp in range(Hp):
            a = full_scr[:, (2 * hp) * W:(2 * hp) * W + W]
            b = full_scr[:, (2 * hp + 1) * W:(2 * hp + 1) * W + W]
            hm = jnp.maximum(a, b)
            pooled = jnp.maximum(
                jnp.dot(hm, E0, preferred_element_type=jnp.float32),
                jnp.dot(hm, E1, preferred_element_type=jnp.float32),
            )
            pool_scr[:, hp * Wp:(hp + 1) * Wp] = pooled
        k_eff = lax.dot_general(
            w_th_ref[...], pool_scr[:C8, :],
            (((0,), (0,)), ((), ())),
            preferred_element_type=jnp.float32,
        )
        kg_ref[0, :C, :] = k_eff.astype(jnp.bfloat16)
        kg_ref[0, C:C + C2, :] = pool_scr[C8:, :].astype(jnp.bfloat16)
        rr = lax.broadcasted_iota(jnp.int32, (8, M), 0)
        kg_ref[0, C + C2:, :] = jnp.where(rr == 0, 1.0, 0.0).astype(jnp.bfloat16)

    kg = pl.pallas_call(
        prep_kernel,
        grid=(B,),
        in_specs=[
            pl.BlockSpec((1, C, N), lambda b: (b, 0, 0)),
            pl.BlockSpec((CP, C), lambda b: (0, 0)),
            pl.BlockSpec((C8, C), lambda b: (0, 0)),
        ],
        out_specs=pl.BlockSpec((1, KG, M), lambda b: (b, 0, 0)),
        out_shape=jax.ShapeDtypeStruct((B, KG, M), jnp.bfloat16),
        scratch_shapes=[
            pltpu.VMEM((CP, N), jnp.float32),
            pltpu.VMEM((CP, M), jnp.float32),
        ],
        compiler_params=pltpu.CompilerParams(
            dimension_semantics=("parallel",),
            vmem_limit_bytes=40 * 1024 * 1024,
        ),
        name="nl_attn_prep",
    )(x_flat, w_pg, w_theta_s)

    def attn_kernel(x_ref, kg_ref, w_o_ref, gamma_ref, o_ref):
        xb = x_ref[0]
        k_eff = kg_ref[0, :C, :]
        g_aug = kg_ref[0, C:, :]
        scores = lax.dot_general(
            xb.astype(jnp.bfloat16), k_eff, (((0,), (0,)), ((), ())),
            preferred_element_type=jnp.float32,
        )
        mx = jnp.max(scores, axis=-1, keepdims=True)
        e = jnp.exp2(scores - mx).astype(jnp.bfloat16)
        o_aug = lax.dot_general(
            g_aug, e, (((1,), (1,)), ((), ())),
            preferred_element_type=jnp.float32,
        )
        o_midT = o_aug[:C2, :]
        s_row = o_aug[C2:C2 + 1, :]
        oT = jnp.dot(w_o_ref[...], o_midT, preferred_element_type=jnp.float32)
        scale = gamma_ref[0] / s_row
        o_ref[0] = oT * scale + xb

    out_flat = pl.pallas_call(
        attn_kernel,
        grid=(B, N // QB),
        in_specs=[
            pl.BlockSpec((1, C, QB), lambda b, q: (b, 0, q)),
            pl.BlockSpec((1, KG, M), lambda b, q: (b, 0, 0)),
            pl.BlockSpec((C, C2), lambda b, q: (0, 0)),
            pl.BlockSpec(memory_space=pltpu.SMEM),
        ],
        out_specs=pl.BlockSpec((1, C, QB), lambda b, q: (b, 0, q)),
        out_shape=jax.ShapeDtypeStruct((B, C, N), jnp.float32),
        compiler_params=pltpu.CompilerParams(
            dimension_semantics=("parallel", "arbitrary"),
            vmem_limit_bytes=48 * 1024 * 1024,
        ),
        name="nl_attn_main",
    )(x_flat, kg, w_o, gamma_arr)

    return out_flat.reshape(B, C, H, W)
